# Initial kernel scaffold; baseline (speedup 1.0000x reference)
import jax
import jax.numpy as jnp
from jax.experimental import pallas as pl


def kernel(z, codebook):
    raise NotImplementedError("write your pallas kernel here")


# speedup vs baseline: 2.3209x; 2.3209x over previous
import jax
import jax.numpy as jnp
from jax.experimental import pallas as pl
from jax.experimental.pallas import tpu as pltpu

N_EMBED = 1024
EMBED_DIM = 64
BETA = 0.25
B = 16
HW = 1024
N_TOK = B * HW


def _vq_kernel(z_ref, cb_ref, out_ref, idx_ref, scalars_ref,
               counts_acc, loss_acc):
    b = pl.program_id(0)

    x = z_ref[0]
    cb = cb_ref[...]

    zf = jnp.transpose(x, (1, 0))

    a = jnp.sum(zf * zf, axis=1, keepdims=True)
    cb_sq = jnp.sum(cb * cb, axis=1)
    m = jnp.dot(zf, cb.T, preferred_element_type=jnp.float32)
    d = (a + cb_sq[None, :]) - 2.0 * m

    lane = jax.lax.broadcasted_iota(jnp.int32, (HW, N_EMBED), 1)
    dmin = jnp.min(d, axis=1, keepdims=True)
    at_min = d == dmin
    idx = jnp.min(jnp.where(at_min, lane, N_EMBED), axis=1).astype(jnp.int32)
    idx_ref[0, 0] = idx

    e = (lane == idx[:, None]).astype(jnp.float32)

    zq_t = jax.lax.dot_general(
        cb, e, (((0,), (1,)), ((), ())),
        preferred_element_type=jnp.float32,
        precision=jax.lax.Precision.HIGHEST)
    out_ref[0] = zq_t

    diff = zq_t - x
    sse = jnp.sum(diff * diff)
    counts = jnp.sum(e, axis=0, keepdims=True)

    @pl.when(b == 0)
    def _init():
        loss_acc[0] = sse
        counts_acc[...] = counts

    @pl.when(b > 0)
    def _accum():
        loss_acc[0] += sse
        counts_acc[...] += counts

    @pl.when(b == B - 1)
    def _finish():
        loss = (1.0 + BETA) * loss_acc[0] / jnp.float32(N_TOK * EMBED_DIM)
        p = counts_acc[...] / jnp.float32(N_TOK)
        ent = jnp.sum(p * jnp.log(p + 1e-10))
        perp = jnp.exp(-ent)
        lane_s = jax.lax.broadcasted_iota(jnp.int32, (1, 128), 1)
        vec = jnp.where(lane_s == 0, loss,
                        jnp.where(lane_s == 1, perp, 0.0))
        scalars_ref[...] = vec


def kernel(z, codebook):
    zr = z.reshape(B, EMBED_DIM, HW)

    out, idx, scalars = pl.pallas_call(
        _vq_kernel,
        grid=(B,),
        in_specs=[
            pl.BlockSpec((1, EMBED_DIM, HW), lambda b: (b, 0, 0)),
            pl.BlockSpec((N_EMBED, EMBED_DIM), lambda b: (0, 0)),
        ],
        out_specs=[
            pl.BlockSpec((1, EMBED_DIM, HW), lambda b: (b, 0, 0)),
            pl.BlockSpec((1, 1, HW), lambda b: (b, 0, 0)),
            pl.BlockSpec((1, 128), lambda b: (0, 0)),
        ],
        out_shape=[
            jax.ShapeDtypeStruct((B, EMBED_DIM, HW), jnp.float32),
            jax.ShapeDtypeStruct((B, 1, HW), jnp.int32),
            jax.ShapeDtypeStruct((1, 128), jnp.float32),
        ],
        scratch_shapes=[
            pltpu.VMEM((1, N_EMBED), jnp.float32),
            pltpu.SMEM((1,), jnp.float32),
        ],
    )(zr, codebook)

    out4 = out.reshape(B, EMBED_DIM, 32, 32)
    loss = scalars[0, 0].reshape(())
    perplexity = scalars[0, 1].reshape(())
    encoding_indices = idx.reshape(N_TOK)
    return (out4, loss, perplexity, encoding_indices)


# speedup vs baseline: 3.9663x; 1.7089x over previous
import jax
import jax.numpy as jnp
from jax.experimental import pallas as pl
from jax.experimental.pallas import tpu as pltpu

N_EMBED = 1024
EMBED_DIM = 64
BETA = 0.25
B = 16
HW = 1024
N_TOK = B * HW


def _vq_kernel(z_ref, cb_ref, out_ref, idx_ref, scalars_ref,
               counts_acc, loss_acc, cbsq_ref):
    b = pl.program_id(0)

    x = z_ref[0]
    cb = cb_ref[...]

    @pl.when(b == 0)
    def _precompute():
        cbsq_ref[...] = jnp.sum(cb * cb, axis=1)[None, :]

    zf = jnp.transpose(x, (1, 0))

    a = jnp.sum(zf * zf, axis=1, keepdims=True)
    cb_sq = cbsq_ref[...]
    m = jnp.dot(zf, cb.T, preferred_element_type=jnp.float32)
    d = (a + cb_sq) - 2.0 * m

    lane = jax.lax.broadcasted_iota(jnp.int32, (HW, N_EMBED), 1)
    dmin = jnp.min(d, axis=1, keepdims=True)
    at_min = d == dmin
    idx = jnp.min(jnp.where(at_min, lane, N_EMBED), axis=1).astype(jnp.int32)
    idx_ref[0, 0] = idx

    e = (lane == idx[:, None]).astype(jnp.bfloat16)
    cb_bf = cb.astype(jnp.bfloat16)

    zq_t = jax.lax.dot_general(
        cb_bf, e, (((0,), (1,)), ((), ())),
        preferred_element_type=jnp.float32)
    out_ref[0] = zq_t

    sse = jnp.sum(dmin)
    ones_row = jnp.ones((8, HW), jnp.bfloat16)
    counts = jnp.dot(ones_row, e, preferred_element_type=jnp.float32)[:1]

    @pl.when(b == 0)
    def _init():
        loss_acc[0] = sse
        counts_acc[...] = counts

    @pl.when(b > 0)
    def _accum():
        loss_acc[0] += sse
        counts_acc[...] += counts

    @pl.when(b == B - 1)
    def _finish():
        loss = (1.0 + BETA) * loss_acc[0] / jnp.float32(N_TOK * EMBED_DIM)
        p = counts_acc[...] / jnp.float32(N_TOK)
        ent = jnp.sum(p * jnp.log(p + 1e-10))
        perp = jnp.exp(-ent)
        lane_s = jax.lax.broadcasted_iota(jnp.int32, (1, 128), 1)
        vec = jnp.where(lane_s == 0, loss,
                        jnp.where(lane_s == 1, perp, 0.0))
        scalars_ref[...] = vec


def kernel(z, codebook):
    zr = z.reshape(B, EMBED_DIM, HW)

    out, idx, scalars = pl.pallas_call(
        _vq_kernel,
        grid=(B,),
        in_specs=[
            pl.BlockSpec((1, EMBED_DIM, HW), lambda b: (b, 0, 0)),
            pl.BlockSpec((N_EMBED, EMBED_DIM), lambda b: (0, 0)),
        ],
        out_specs=[
            pl.BlockSpec((1, EMBED_DIM, HW), lambda b: (b, 0, 0)),
            pl.BlockSpec((1, 1, HW), lambda b: (b, 0, 0)),
            pl.BlockSpec((1, 128), lambda b: (0, 0)),
        ],
        out_shape=[
            jax.ShapeDtypeStruct((B, EMBED_DIM, HW), jnp.float32),
            jax.ShapeDtypeStruct((B, 1, HW), jnp.int32),
            jax.ShapeDtypeStruct((1, 128), jnp.float32),
        ],
        scratch_shapes=[
            pltpu.VMEM((1, N_EMBED), jnp.float32),
            pltpu.SMEM((1,), jnp.float32),
            pltpu.VMEM((1, N_EMBED), jnp.float32),
        ],
    )(zr, codebook)

    out4 = out.reshape(B, EMBED_DIM, 32, 32)
    loss = scalars[0, 0].reshape(())
    perplexity = scalars[0, 1].reshape(())
    encoding_indices = idx.reshape(N_TOK)
    return (out4, loss, perplexity, encoding_indices)
